# baseline (speedup 1.0000x reference)
import functools

import jax
import jax.numpy as jnp
from jax import lax
from jax.experimental import pallas as pl
from jax.experimental.pallas import tpu as pltpu
from jax.experimental.pallas import tpu_sc as plsc

B, C, H, W = 16, 384, 56, 56
P = B * H * W
NW = 32
PPW = P // NW
GP = 56
NG = PPW // GP
CH16 = C // 16


def _exchange_body(x0_hbm, x1_hbm, w0_hbm, w1_hbm, o1_hbm, o2_hbm,
                   b0, b1, w0v, w1v, m1v, m2v):
    wid = lax.axis_index("s") * 2 + lax.axis_index("c")

    pltpu.sync_copy(w0_hbm, w0v)
    pltpu.sync_copy(w1_hbm, w1v)

    mn = jnp.abs(w0v[pl.ds(0, 16)])
    mx = mn
    for j in range(1, CH16):
        a = jnp.abs(w0v[pl.ds(j * 16, 16)])
        mn = jnp.minimum(mn, a)
        mx = jnp.maximum(mx, a)
    dnums = lax.GatherDimensionNumbers(
        offset_dims=(), collapsed_slice_dims=(0,), start_index_map=(0,))

    def shuffle(v, perm):
        return lax.gather(v, perm[:, None], dnums, slice_sizes=(1,),
                          mode=lax.GatherScatterMode.PROMISE_IN_BOUNDS)

    idx = lax.iota(jnp.int32, 16)
    for k in (8, 4, 2, 1):
        perm = jnp.bitwise_xor(idx, k)
        mn = jnp.minimum(mn, shuffle(mn, perm))
        mx = jnp.maximum(mx, shuffle(mx, perm))
    thrv = mn + 0.05 * (mx - mn)

    one = jnp.full((16,), 1.0, jnp.float32)
    zero = jnp.full((16,), 0.0, jnp.float32)
    for j in range(CH16):
        sl = pl.ds(j * 16, 16)
        m1v[sl] = jnp.where(jnp.abs(w0v[sl]) >= thrv, one, zero)
        m2v[sl] = jnp.where(jnp.abs(w1v[sl]) >= thrv, one, zero)

    def chunk(g, _):
        base = wid * PPW + g * GP
        pltpu.sync_copy(x0_hbm.at[pl.ds(base, GP)], b0)
        pltpu.sync_copy(x1_hbm.at[pl.ds(base, GP)], b1)

        def pixel(p, _):
            for j in range(CH16):
                sl = pl.ds(j * 16, 16)
                a = b0[p, sl]
                b = b1[p, sl]
                prod = a * b
                b0[p, sl] = jnp.where(m1v[sl] > zero, a, prod)
                b1[p, sl] = jnp.where(m2v[sl] > zero, b, prod)
            return 0

        lax.fori_loop(0, GP, pixel, 0)
        pltpu.sync_copy(b0, o1_hbm.at[pl.ds(base, GP)])
        pltpu.sync_copy(b1, o2_hbm.at[pl.ds(base, GP)])
        return 0

    lax.fori_loop(0, NG, chunk, 0)


def kernel(x0, x1, insnorm_weight0, insnorm_weight1, threshold):
    del threshold
    x0t = jnp.transpose(x0, (0, 2, 3, 1)).reshape(P, C)
    x1t = jnp.transpose(x1, (0, 2, 3, 1)).reshape(P, C)

    f32 = jnp.float32
    run = pl.kernel(
        _exchange_body,
        out_type=(
            jax.ShapeDtypeStruct((P, C), f32),
            jax.ShapeDtypeStruct((P, C), f32),
        ),
        mesh=plsc.VectorSubcoreMesh(core_axis_name="c", subcore_axis_name="s"),
        scratch_types=(
            pltpu.VMEM((GP, C), f32),
            pltpu.VMEM((GP, C), f32),
            pltpu.VMEM((C,), f32),
            pltpu.VMEM((C,), f32),
            pltpu.VMEM((C,), f32),
            pltpu.VMEM((C,), f32),
        ),
    )
    o1, o2 = run(x0t, x1t, insnorm_weight0, insnorm_weight1)
    o1 = jnp.transpose(o1.reshape(B, H, W, C), (0, 3, 1, 2))
    o2 = jnp.transpose(o2.reshape(B, H, W, C), (0, 3, 1, 2))
    return (o1, o2)


# speedup vs baseline: 2.9001x; 2.9001x over previous
import jax
import jax.numpy as jnp
from jax import lax
from jax.experimental import pallas as pl
from jax.experimental.pallas import tpu as pltpu
from jax.experimental.pallas import tpu_sc as plsc

B, C, H, W = 16, 384, 56, 56
P = B * H * W
NW = 32
PPW = P // NW
GP = 16
NG = PPW // GP
CH16 = C // 16
UNROLL = 4


def _exchange_body(x0_hbm, x1_hbm, w0_hbm, w1_hbm, o1_hbm, o2_hbm,
                   bx0_0, bx0_1, bx1_0, bx1_1, bo1_0, bo1_1, bo2_0, bo2_1,
                   w0v, w1v, m1v, m2v,
                   sx0_0, sx0_1, sx1_0, sx1_1, so1_0, so1_1, so2_0, so2_1):
    bx0 = (bx0_0, bx0_1)
    bx1 = (bx1_0, bx1_1)
    bo1 = (bo1_0, bo1_1)
    bo2 = (bo2_0, bo2_1)
    sx0 = (sx0_0, sx0_1)
    sx1 = (sx1_0, sx1_1)
    so1 = (so1_0, so1_1)
    so2 = (so2_0, so2_1)

    wid = lax.axis_index("s") * 2 + lax.axis_index("c")
    base_w = wid * PPW

    pltpu.sync_copy(w0_hbm, w0v)
    pltpu.sync_copy(w1_hbm, w1v)

    mn = jnp.abs(w0v[pl.ds(0, 16)])
    mx = mn
    for j in range(1, CH16):
        a = jnp.abs(w0v[pl.ds(j * 16, 16)])
        mn = jnp.minimum(mn, a)
        mx = jnp.maximum(mx, a)
    dnums = lax.GatherDimensionNumbers(
        offset_dims=(), collapsed_slice_dims=(0,), start_index_map=(0,))

    def shuffle(v, perm):
        return lax.gather(v, perm[:, None], dnums, slice_sizes=(1,),
                          mode=lax.GatherScatterMode.PROMISE_IN_BOUNDS)

    idx = lax.iota(jnp.int32, 16)
    for k in (8, 4, 2, 1):
        perm = jnp.bitwise_xor(idx, k)
        mn = jnp.minimum(mn, shuffle(mn, perm))
        mx = jnp.maximum(mx, shuffle(mx, perm))
    thrv = mn + 0.05 * (mx - mn)

    one = jnp.full((16,), 1.0, jnp.float32)
    zero = jnp.full((16,), 0.0, jnp.float32)
    for j in range(CH16):
        sl = pl.ds(j * 16, 16)
        m1v[sl] = jnp.where(jnp.abs(w0v[sl]) >= thrv, one, zero)
        m2v[sl] = jnp.where(jnp.abs(w1v[sl]) >= thrv, one, zero)

    def in_copies(g, s):
        base = base_w + g * GP
        return (
            pltpu.make_async_copy(x0_hbm.at[pl.ds(base, GP)], bx0[s], sx0[s]),
            pltpu.make_async_copy(x1_hbm.at[pl.ds(base, GP)], bx1[s], sx1[s]),
        )

    def out_copies(g, s):
        base = base_w + g * GP
        return (
            pltpu.make_async_copy(bo1[s], o1_hbm.at[pl.ds(base, GP)], so1[s]),
            pltpu.make_async_copy(bo2[s], o2_hbm.at[pl.ds(base, GP)], so2[s]),
        )

    def compute(s):
        bi0, bi1, bu1, bu2 = bx0[s], bx1[s], bo1[s], bo2[s]
        for j in range(CH16):
            sl = pl.ds(j * 16, 16)
            m1 = m1v[sl] > zero
            m2 = m2v[sl] > zero

            @plsc.parallel_loop(0, GP, 1, unroll=UNROLL)
            def _(p):
                a = bi0[p, sl]
                b = bi1[p, sl]
                prod = a * b
                bu1[p, sl] = jnp.where(m1, a, prod)
                bu2[p, sl] = jnp.where(m2, b, prod)

    for c in in_copies(0, 0):
        c.start()
    for c in in_copies(1, 1):
        c.start()

    def pair(t, _):
        for s in (0, 1):
            g = 2 * t + s
            for c in in_copies(g, s):
                c.wait()

            @pl.when(t > 0)
            def _():
                for c in out_copies(g - 2, s):
                    c.wait()

            compute(s)
            for c in out_copies(g, s):
                c.start()

            @pl.when(g + 2 < NG)
            def _():
                for c in in_copies(g + 2, s):
                    c.start()
        return 0

    lax.fori_loop(0, NG // 2, pair, 0)

    for s in (0, 1):
        for c in out_copies(NG - 2 + s, s):
            c.wait()


def kernel(x0, x1, insnorm_weight0, insnorm_weight1, threshold):
    del threshold
    x0t = jnp.transpose(x0, (0, 2, 3, 1)).reshape(P, C)
    x1t = jnp.transpose(x1, (0, 2, 3, 1)).reshape(P, C)

    f32 = jnp.float32
    buf = pltpu.VMEM((GP, C), f32)
    vec = pltpu.VMEM((C,), f32)
    sem = pltpu.SemaphoreType.DMA
    run = pl.kernel(
        _exchange_body,
        out_type=(
            jax.ShapeDtypeStruct((P, C), f32),
            jax.ShapeDtypeStruct((P, C), f32),
        ),
        mesh=plsc.VectorSubcoreMesh(core_axis_name="c", subcore_axis_name="s"),
        scratch_types=(buf,) * 8 + (vec,) * 4 + (sem,) * 8,
    )
    o1, o2 = run(x0t, x1t, insnorm_weight0, insnorm_weight1)
    o1 = jnp.transpose(o1.reshape(B, H, W, C), (0, 3, 1, 2))
    o2 = jnp.transpose(o2.reshape(B, H, W, C), (0, 3, 1, 2))
    return (o1, o2)


# speedup vs baseline: 3.6073x; 1.2439x over previous
import jax
import jax.numpy as jnp
from jax import lax
from jax.experimental import pallas as pl
from jax.experimental.pallas import tpu as pltpu
from jax.experimental.pallas import tpu_sc as plsc

B, C, H, W = 16, 384, 56, 56
P = B * H * W
NW = 32
PPW = P // NW
GP = 16
NG = PPW // GP
CH16 = C // 16
UNROLL = 8


def _exchange_body(x0_hbm, x1_hbm, w0_hbm, w1_hbm, o1_hbm, o2_hbm,
                   bx0_0, bx0_1, bx1_0, bx1_1, bo1_0, bo1_1, bo2_0, bo2_1,
                   w0v, w1v, m1v, m2v,
                   sx0_0, sx0_1, sx1_0, sx1_1, so1_0, so1_1, so2_0, so2_1):
    bx0 = (bx0_0, bx0_1)
    bx1 = (bx1_0, bx1_1)
    bo1 = (bo1_0, bo1_1)
    bo2 = (bo2_0, bo2_1)
    sx0 = (sx0_0, sx0_1)
    sx1 = (sx1_0, sx1_1)
    so1 = (so1_0, so1_1)
    so2 = (so2_0, so2_1)

    wid = lax.axis_index("s") * 2 + lax.axis_index("c")
    base_w = wid * PPW

    pltpu.sync_copy(w0_hbm, w0v)
    pltpu.sync_copy(w1_hbm, w1v)

    mn = jnp.abs(w0v[pl.ds(0, 16)])
    mx = mn
    for j in range(1, CH16):
        a = jnp.abs(w0v[pl.ds(j * 16, 16)])
        mn = jnp.minimum(mn, a)
        mx = jnp.maximum(mx, a)
    dnums = lax.GatherDimensionNumbers(
        offset_dims=(), collapsed_slice_dims=(0,), start_index_map=(0,))

    def shuffle(v, perm):
        return lax.gather(v, perm[:, None], dnums, slice_sizes=(1,),
                          mode=lax.GatherScatterMode.PROMISE_IN_BOUNDS)

    idx = lax.iota(jnp.int32, 16)
    for k in (8, 4, 2, 1):
        perm = jnp.bitwise_xor(idx, k)
        mn = jnp.minimum(mn, shuffle(mn, perm))
        mx = jnp.maximum(mx, shuffle(mx, perm))
    thrv = mn + 0.05 * (mx - mn)

    one = jnp.full((16,), 1.0, jnp.float32)
    zero = jnp.full((16,), 0.0, jnp.float32)
    for j in range(CH16):
        sl = pl.ds(j * 16, 16)
        m1v[sl] = jnp.where(jnp.abs(w0v[sl]) >= thrv, one, zero)
        m2v[sl] = jnp.where(jnp.abs(w1v[sl]) >= thrv, one, zero)

    def in_copies(g, s):
        base = base_w + g * GP
        return (
            pltpu.make_async_copy(x0_hbm.at[pl.ds(base, GP)], bx0[s], sx0[s]),
            pltpu.make_async_copy(x1_hbm.at[pl.ds(base, GP)], bx1[s], sx1[s]),
        )

    def out_copies(g, s):
        base = base_w + g * GP
        return (
            pltpu.make_async_copy(bo1[s], o1_hbm.at[pl.ds(base, GP)], so1[s]),
            pltpu.make_async_copy(bo2[s], o2_hbm.at[pl.ds(base, GP)], so2[s]),
        )

    def compute(s):
        bi0, bi1, bu1, bu2 = bx0[s], bx1[s], bo1[s], bo2[s]
        for j in range(CH16):
            sl = pl.ds(j * 16, 16)
            m1 = m1v[sl] > zero
            m2 = m2v[sl] > zero

            @plsc.parallel_loop(0, GP, 1, unroll=UNROLL)
            def _(p):
                a = bi0[p, sl]
                b = bi1[p, sl]
                prod = a * b
                bu1[p, sl] = jnp.where(m1, a, prod)
                bu2[p, sl] = jnp.where(m2, b, prod)

    for c in in_copies(0, 0):
        c.start()
    for c in in_copies(1, 1):
        c.start()

    def pair(t, _):
        for s in (0, 1):
            g = 2 * t + s
            for c in in_copies(g, s):
                c.wait()

            @pl.when(t > 0)
            def _():
                for c in out_copies(g - 2, s):
                    c.wait()

            compute(s)
            for c in out_copies(g, s):
                c.start()

            @pl.when(g + 2 < NG)
            def _():
                for c in in_copies(g + 2, s):
                    c.start()
        return 0

    lax.fori_loop(0, NG // 2, pair, 0)

    for s in (0, 1):
        for c in out_copies(NG - 2 + s, s):
            c.wait()


def kernel(x0, x1, insnorm_weight0, insnorm_weight1, threshold):
    del threshold
    x0t = jnp.transpose(x0, (0, 2, 3, 1)).reshape(P, C)
    x1t = jnp.transpose(x1, (0, 2, 3, 1)).reshape(P, C)

    f32 = jnp.float32
    buf = pltpu.VMEM((GP, C), f32)
    vec = pltpu.VMEM((C,), f32)
    sem = pltpu.SemaphoreType.DMA
    run = pl.kernel(
        _exchange_body,
        out_type=(
            jax.ShapeDtypeStruct((P, C), f32),
            jax.ShapeDtypeStruct((P, C), f32),
        ),
        mesh=plsc.VectorSubcoreMesh(core_axis_name="c", subcore_axis_name="s"),
        scratch_types=(buf,) * 8 + (vec,) * 4 + (sem,) * 8,
    )
    o1, o2 = run(x0t, x1t, insnorm_weight0, insnorm_weight1)
    o1 = jnp.transpose(o1.reshape(B, H, W, C), (0, 3, 1, 2))
    o2 = jnp.transpose(o2.reshape(B, H, W, C), (0, 3, 1, 2))
    return (o1, o2)
